# baseline (speedup 1.0000x reference)
import jax
import jax.numpy as jnp
from jax import lax
from jax.experimental import pallas as pl
from jax.experimental.pallas import tpu as pltpu
from jax.experimental.pallas import tpu_sc as plsc

_B, _Y, _X = 16, 4096, 512
_NEG = -10000
_L = 16
_UNROLL = 4


def _sc_hist(dur_hbm, olen_hbm, ilen_hbm, d_hbm, dbuf, ubuf, obuf, lbuf,
             ibuf, sem):
    w = lax.axis_index("s")

    @pl.when(w < _B)
    def _():
        row = w
        in_cp = pltpu.async_copy(dur_hbm.at[row], dbuf, sem)
        pltpu.sync_copy(olen_hbm, lbuf)
        pltpu.sync_copy(ilen_hbm, ibuf)

        lane = lax.iota(jnp.int32, _L)
        out_len = jnp.max(jnp.where(lane == row, lbuf[...], 0))
        max_in = jnp.max(ibuf[...])

        ubuf[pl.ds(0, _L)] = jnp.zeros((_L,), jnp.int32)
        in_cp.wait()

        def search(i, carry):
            for u in range(_UNROLL):
                j = i * _UNROLL + u
                x = j * _L + lane
                lo = jnp.zeros((_L,), jnp.int32)
                for k in range(11, -1, -1):
                    cand = lo + (1 << k)
                    probe = plsc.load_gather(dbuf, [cand - 1])
                    lo = jnp.where(probe <= x, cand, lo)
                probe = plsc.load_gather(dbuf, [lo])
                lo = lo + (probe <= x).astype(jnp.int32)
                ubuf[pl.ds(j * _L + _L, _L)] = lo
            return carry

        lax.fori_loop(0, _X // (_L * _UNROLL), search, 0)

        def diff(j, carry):
            base = j * _L
            cur = ubuf[pl.ds(base + _L, _L)]
            prev = plsc.load_gather(ubuf, [base + _L - 1 + lane])
            d = jnp.minimum(cur, out_len) - jnp.minimum(prev, out_len)
            x = base + lane
            obuf[pl.ds(base, _L)] = jnp.where(x < max_in, d, 0)
            return carry

        lax.fori_loop(0, _X // _L, diff, 0)

        pltpu.sync_copy(obuf, d_hbm.at[row])


def _tc_weights(dur_ref, olen_ref, out_ref):
    pos = lax.broadcasted_iota(jnp.int32, (_B, _Y), 1)
    mask = pos < olen_ref[...]
    dur = dur_ref[...]
    out_ref[...] = jnp.where(mask, dur, dur + _NEG)


@jax.jit
def kernel(duration, output_length, input_length):
    mesh = plsc.VectorSubcoreMesh(
        core_axis_name="c", subcore_axis_name="s", num_cores=1)
    hist = pl.kernel(
        _sc_hist,
        out_type=jax.ShapeDtypeStruct((_B, _X), jnp.int32),
        mesh=mesh,
        compiler_params=pltpu.CompilerParams(needs_layout_passes=False),
        scratch_types=[
            pltpu.VMEM((_Y,), jnp.int32),
            pltpu.VMEM((_X + _L,), jnp.int32),
            pltpu.VMEM((_X,), jnp.int32),
            pltpu.VMEM((_L,), jnp.int32),
            pltpu.VMEM((_L,), jnp.int32),
            pltpu.SemaphoreType.DMA,
        ],
    )
    durations = hist(duration, output_length, input_length)

    weights = pl.pallas_call(
        _tc_weights,
        out_shape=jax.ShapeDtypeStruct((_B, _Y), jnp.int32),
    )(duration, output_length.reshape(_B, 1))

    return (weights, durations)


# speedup vs baseline: 1.1019x; 1.1019x over previous
import jax
import jax.numpy as jnp
from jax import lax
from jax.experimental import pallas as pl
from jax.experimental.pallas import tpu as pltpu
from jax.experimental.pallas import tpu_sc as plsc

_B, _Y, _X = 16, 4096, 512
_NEG = -10000
_L = 16
_UNROLL = 8


def _sc_hist(dur_hbm, olen_hbm, ilen_hbm, d_hbm, dbuf, obuf, lbuf, ibuf, sem):
    w = lax.axis_index("s")

    @pl.when(w < _B)
    def _():
        row = w
        in_cp = pltpu.async_copy(dur_hbm.at[row], dbuf.at[pl.ds(0, _Y)], sem)
        pltpu.sync_copy(olen_hbm, lbuf)
        pltpu.sync_copy(ilen_hbm, ibuf)

        lane = lax.iota(jnp.int32, _L)
        out_len = jnp.max(jnp.where(lane == row, lbuf[...], 0))
        max_in = jnp.max(ibuf[...])

        zeros = jnp.zeros((_L,), jnp.int32)

        def zero_o(j, carry):
            obuf[pl.ds(j * _L, _L)] = zeros
            return carry

        lax.fori_loop(0, _X // _L, zero_o, 0)
        in_cp.wait()

        lanep1 = lane + 1

        def step(base, is_final):
            val = dbuf[pl.ds(base, _L)]
            nxt = plsc.load_gather(dbuf, [base + lanep1])
            is_last = val != nxt
            if is_final:
                is_last = is_last | (lane == _L - 1)
            m1 = jnp.minimum(base + lanep1, out_len)
            plsc.addupdate_scatter(obuf, [val], m1, mask=is_last)
            if is_final:
                is_last = is_last & (lane != _L - 1)
            plsc.addupdate_scatter(obuf, [nxt], -m1, mask=is_last)

        def pass_row(i, carry):
            for u in range(_UNROLL):
                step((i * _UNROLL + u) * _L, False)
            return carry

        lax.fori_loop(0, _Y // (_L * _UNROLL) - 1, pass_row, 0)
        for u in range(_UNROLL - 1):
            step(_Y - _UNROLL * _L + u * _L, False)
        step(_Y - _L, True)

        def mask_o(j, carry):
            base = j * _L
            v = obuf[pl.ds(base, _L)]
            obuf[pl.ds(base, _L)] = jnp.where(base + lane < max_in, v, 0)
            return carry

        lax.fori_loop(0, _X // _L, mask_o, 0)

        pltpu.sync_copy(obuf, d_hbm.at[row])


def _tc_weights(dur_ref, olen_ref, out_ref):
    pos = lax.broadcasted_iota(jnp.int32, (_B, _Y), 1)
    mask = pos < olen_ref[...]
    dur = dur_ref[...]
    out_ref[...] = jnp.where(mask, dur, dur + _NEG)


@jax.jit
def kernel(duration, output_length, input_length):
    mesh = plsc.VectorSubcoreMesh(
        core_axis_name="c", subcore_axis_name="s", num_cores=1)
    hist = pl.kernel(
        _sc_hist,
        out_type=jax.ShapeDtypeStruct((_B, _X), jnp.int32),
        mesh=mesh,
        compiler_params=pltpu.CompilerParams(needs_layout_passes=False),
        scratch_types=[
            pltpu.VMEM((_Y + _L,), jnp.int32),
            pltpu.VMEM((_X,), jnp.int32),
            pltpu.VMEM((_L,), jnp.int32),
            pltpu.VMEM((_L,), jnp.int32),
            pltpu.SemaphoreType.DMA,
        ],
    )
    durations = hist(duration, output_length, input_length)

    weights = pl.pallas_call(
        _tc_weights,
        out_shape=jax.ShapeDtypeStruct((_B, _Y), jnp.int32),
    )(duration, output_length.reshape(_B, 1))

    return (weights, durations)


# speedup vs baseline: 1.1022x; 1.0003x over previous
import jax
import jax.numpy as jnp
from jax import lax
from jax.experimental import pallas as pl
from jax.experimental.pallas import tpu as pltpu
from jax.experimental.pallas import tpu_sc as plsc

_B, _Y, _X = 16, 4096, 512
_NEG = -10000
_L = 16
_UNROLL = 8


def _sc_hist(dur_hbm, olen_hbm, ilen_hbm, d_hbm, dbuf, obuf, lbuf, ibuf, sem):
    w = lax.axis_index("s")

    @pl.when(w < _B)
    def _():
        row = w
        in_cp = pltpu.async_copy(dur_hbm.at[row], dbuf.at[pl.ds(0, _Y)], sem)
        pltpu.sync_copy(olen_hbm, lbuf)
        pltpu.sync_copy(ilen_hbm, ibuf)

        lane = lax.iota(jnp.int32, _L)
        out_len = jnp.max(jnp.where(lane == row, lbuf[...], 0))
        max_in = jnp.max(ibuf[...])

        zeros = jnp.zeros((_L,), jnp.int32)

        def zero_o(j, carry):
            obuf[pl.ds(j * _L, _L)] = zeros
            return carry

        lax.fori_loop(0, _X // _L, zero_o, 0)
        in_cp.wait()

        lanep1 = lane + 1

        def step(base, is_final):
            val = dbuf[pl.ds(base, _L)]
            nxt = dbuf[pl.ds(base + 1, _L)]
            is_last = val != nxt
            if is_final:
                is_last = is_last | (lane == _L - 1)
            m1 = jnp.minimum(base + lanep1, out_len)
            plsc.addupdate_scatter(obuf, [val], m1, mask=is_last)
            if is_final:
                is_last = is_last & (lane != _L - 1)
            plsc.addupdate_scatter(obuf, [nxt], -m1, mask=is_last)

        def pass_row(i, carry):
            for u in range(_UNROLL):
                step((i * _UNROLL + u) * _L, False)
            return carry

        lax.fori_loop(0, _Y // (_L * _UNROLL) - 1, pass_row, 0)
        for u in range(_UNROLL - 1):
            step(_Y - _UNROLL * _L + u * _L, False)
        step(_Y - _L, True)

        def mask_o(j, carry):
            base = j * _L
            v = obuf[pl.ds(base, _L)]
            obuf[pl.ds(base, _L)] = jnp.where(base + lane < max_in, v, 0)
            return carry

        lax.fori_loop(0, _X // _L, mask_o, 0)

        pltpu.sync_copy(obuf, d_hbm.at[row])


def _tc_weights(dur_ref, olen_ref, out_ref):
    pos = lax.broadcasted_iota(jnp.int32, (_B, _Y), 1)
    mask = pos < olen_ref[...]
    dur = dur_ref[...]
    out_ref[...] = jnp.where(mask, dur, dur + _NEG)


@jax.jit
def kernel(duration, output_length, input_length):
    mesh = plsc.VectorSubcoreMesh(
        core_axis_name="c", subcore_axis_name="s", num_cores=1)
    hist = pl.kernel(
        _sc_hist,
        out_type=jax.ShapeDtypeStruct((_B, _X), jnp.int32),
        mesh=mesh,
        compiler_params=pltpu.CompilerParams(needs_layout_passes=False),
        scratch_types=[
            pltpu.VMEM((_Y + _L,), jnp.int32),
            pltpu.VMEM((_X,), jnp.int32),
            pltpu.VMEM((_L,), jnp.int32),
            pltpu.VMEM((_L,), jnp.int32),
            pltpu.SemaphoreType.DMA,
        ],
    )
    durations = hist(duration, output_length, input_length)

    weights = pl.pallas_call(
        _tc_weights,
        out_shape=jax.ShapeDtypeStruct((_B, _Y), jnp.int32),
    )(duration, output_length.reshape(_B, 1))

    return (weights, durations)
